# baseline (speedup 1.0000x reference)
import functools

import jax
import jax.numpy as jnp
from jax import lax
from jax.experimental import pallas as pl
from jax.experimental.pallas import tpu as pltpu
from jax.experimental.pallas import tpu_sc as plsc

N = 10000
E = 320000
D = 128
H = 128
C = 64

NC = 2
NS = 16
NW = NC * NS

EPT = E // NW
K = 80
NCH = EPT // K
IRING = 8
GRING = 4
RCH = 80
NRCH = N // RCH
RROUND = -(-NRCH // NS)

_MM_BLK = 1000


def _seg_partials(F):
  mesh = plsc.VectorSubcoreMesh(core_axis_name="c", subcore_axis_name="s")

  scratch = []
  for _ in range(IRING):
    scratch += [
        pltpu.VMEM((K,), jnp.int32),
        pltpu.VMEM((K,), jnp.int32),
        pltpu.SemaphoreType.DMA,
        pltpu.SemaphoreType.DMA,
    ]
  for _ in range(GRING):
    scratch += [
        pltpu.VMEM((K, F), jnp.float32),
        pltpu.SemaphoreType.DMA,
        pltpu.SemaphoreType.DMA,
    ]
  scratch.append(pltpu.VMEM_SHARED((N, F), jnp.float32))

  @functools.partial(
      pl.kernel,
      out_type=jax.ShapeDtypeStruct((NC, N, F), jnp.float32),
      mesh=mesh,
      scratch_types=scratch,
  )
  def seg(x_hbm, src_hbm, dst_hbm, zrows_hbm, out_hbm, *bufs):
    sidx = [bufs[4 * b + 0] for b in range(IRING)]
    didx = [bufs[4 * b + 1] for b in range(IRING)]
    ssem = [bufs[4 * b + 2] for b in range(IRING)]
    dsem = [bufs[4 * b + 3] for b in range(IRING)]
    g0 = 4 * IRING
    rows = [bufs[g0 + 3 * b] for b in range(GRING)]
    gsem = [bufs[g0 + 3 * b + 1] for b in range(GRING)]
    csem = [bufs[g0 + 3 * b + 2] for b in range(GRING)]
    acc = bufs[-1]

    c = lax.axis_index("c")
    s = lax.axis_index("s")
    wid = s * NC + c
    ebase = wid * EPT

    def issue_idx(jc, ib):
      off = ebase + jc * K
      pltpu.async_copy(src_hbm.at[pl.ds(off, K)], sidx[ib], ssem[ib])
      pltpu.async_copy(dst_hbm.at[pl.ds(off, K)], didx[ib], dsem[ib])

    def issue_gather(ib, gb):
      pltpu.make_async_copy(src_hbm.at[pl.ds(0, K)], sidx[ib],
                            ssem[ib]).wait()
      pltpu.async_copy(x_hbm.at[sidx[ib]], rows[gb], gsem[gb])

    def drain_scatter(ib, gb):
      pltpu.make_async_copy(rows[gb], acc.at[didx[ib]], csem[gb]).wait()

    for b in range(IRING - 2):
      issue_idx(b, b)

    pltpu.sync_copy(zrows_hbm, rows[0].at[pl.ds(0, RCH)])

    def zero_chunk(jj, carry):
      j = s + NS * jj

      @pl.when(j < NRCH)
      def _():
        pltpu.sync_copy(rows[0].at[pl.ds(0, RCH)], acc.at[pl.ds(j * RCH, RCH)])

      return carry

    lax.fori_loop(0, RROUND, zero_chunk, 0)

    for b in range(GRING - 2):
      issue_gather(b, b)
    plsc.subcore_barrier()

    def step(jj, carry):
      for b in range(IRING):
        jc = jj * IRING + b
        gb = b % GRING

        @pl.when((jc >= 2) & (jc - 2 < NCH))
        def _(jc=jc, b=b, gb=gb):
          drain_scatter((b + IRING - 2) % IRING, (gb + GRING - 2) % GRING)

        @pl.when(jc + IRING - 2 < NCH)
        def _(jc=jc, b=b):
          issue_idx(jc + IRING - 2, (b + IRING - 2) % IRING)

        @pl.when(jc + GRING - 2 < NCH)
        def _(jc=jc, b=b, gb=gb):
          issue_gather((b + GRING - 2) % IRING, (gb + GRING - 2) % GRING)

        @pl.when(jc < NCH)
        def _(jc=jc, b=b, gb=gb):
          pltpu.make_async_copy(x_hbm.at[sidx[b]], rows[gb], gsem[gb]).wait()
          pltpu.make_async_copy(dst_hbm.at[pl.ds(0, K)], didx[b],
                                dsem[b]).wait()
          pltpu.async_copy(rows[gb], acc.at[didx[b]], csem[gb], add=True)

      return carry

    lax.fori_loop(0, -(-(NCH + 2) // IRING), step, 0)
    plsc.subcore_barrier()

    def wb_chunk(jj, carry):
      j = s + NS * jj

      @pl.when(j < NRCH)
      def _():
        r0 = j * RCH
        pltpu.sync_copy(acc.at[pl.ds(r0, RCH)], rows[0].at[pl.ds(0, RCH)])
        pltpu.sync_copy(rows[0].at[pl.ds(0, RCH)],
                        out_hbm.at[c, pl.ds(r0, RCH)])

      return carry

    lax.fori_loop(0, RROUND, wb_chunk, 0)

  return seg


_seg128 = _seg_partials(H)


def _fused_mm_body(x_ref, p_ref, b_ref, w_ref, o_ref, *, relu):
  z = x_ref[...] + p_ref[0] + p_ref[1]
  y = jnp.dot(z, w_ref[...], preferred_element_type=jnp.float32) + b_ref[...]
  o_ref[...] = jnp.maximum(y, 0.0) if relu else y


def _fused_mm(x, p, b, w, relu):
  nblk = N // _MM_BLK
  din = x.shape[1]
  dout = w.shape[1]
  return pl.pallas_call(
      functools.partial(_fused_mm_body, relu=relu),
      grid=(nblk,),
      in_specs=[
          pl.BlockSpec((_MM_BLK, din), lambda i: (i, 0)),
          pl.BlockSpec((NC, _MM_BLK, din), lambda i: (0, i, 0)),
          pl.BlockSpec((1, dout), lambda i: (0, 0)),
          pl.BlockSpec((din, dout), lambda i: (0, 0)),
      ],
      out_specs=pl.BlockSpec((_MM_BLK, dout), lambda i: (i, 0)),
      out_shape=jax.ShapeDtypeStruct((N, dout), jnp.float32),
  )(x, p, b.reshape(1, dout), w)


def kernel(features, adj, W1, b1, W2, b2):
  src = adj[0]
  dst = adj[1]
  zrows = jnp.zeros((RCH, H), jnp.float32)

  p1 = _seg128(features, src, dst, zrows)
  z1 = _fused_mm(features, p1, b1, W1, relu=True)
  p2 = _seg128(z1, src, dst, zrows)
  out = _fused_mm(z1, p2, b2, W2, relu=False)
  return out


# speedup vs baseline: 1.0749x; 1.0749x over previous
import functools

import jax
import jax.numpy as jnp
from jax import lax
from jax.experimental import pallas as pl
from jax.experimental.pallas import tpu as pltpu
from jax.experimental.pallas import tpu_sc as plsc

N = 10000
E = 320000
D = 128
H = 128
C = 64

NC = 2
NS = 16
NW = NC * NS

EPT = E // NW
K = 80
NCH = EPT // K
IRING = 8
GRING = 4
RCH = 80
NRCH = N // RCH
RROUND = -(-NRCH // NS)

_MM_BLK = 1000


def _seg_partials(F):
  mesh = plsc.VectorSubcoreMesh(core_axis_name="c", subcore_axis_name="s")

  scratch = []
  for _ in range(IRING):
    scratch += [
        pltpu.VMEM((K,), jnp.int32),
        pltpu.VMEM((K,), jnp.int32),
        pltpu.SemaphoreType.DMA,
        pltpu.SemaphoreType.DMA,
    ]
  for _ in range(GRING):
    scratch += [
        pltpu.VMEM((K, F), jnp.float32),
        pltpu.SemaphoreType.DMA,
    ]
  scratch.append(pltpu.VMEM_SHARED((N, F), jnp.float32))

  @functools.partial(
      pl.kernel,
      out_type=jax.ShapeDtypeStruct((NC, N, F), jnp.float32),
      mesh=mesh,
      scratch_types=scratch,
  )
  def seg(x_hbm, src_hbm, dst_hbm, zrows_hbm, out_hbm, *bufs):
    sidx = [bufs[4 * b + 0] for b in range(IRING)]
    didx = [bufs[4 * b + 1] for b in range(IRING)]
    ssem = [bufs[4 * b + 2] for b in range(IRING)]
    dsem = [bufs[4 * b + 3] for b in range(IRING)]
    g0 = 4 * IRING
    rows = [bufs[g0 + 2 * b] for b in range(GRING)]
    gsem = [bufs[g0 + 2 * b + 1] for b in range(GRING)]
    acc = bufs[-1]

    c = lax.axis_index("c")
    s = lax.axis_index("s")
    wid = s * NC + c
    ebase = wid * EPT

    def issue_idx(jc, ib):
      off = ebase + jc * K
      pltpu.async_copy(src_hbm.at[pl.ds(off, K)], sidx[ib], ssem[ib])
      pltpu.async_copy(dst_hbm.at[pl.ds(off, K)], didx[ib], dsem[ib])

    def issue_gather(ib, gb):
      pltpu.make_async_copy(src_hbm.at[pl.ds(0, K)], sidx[ib],
                            ssem[ib]).wait()
      pltpu.async_copy(x_hbm.at[sidx[ib]], rows[gb], gsem[gb])

    for b in range(IRING - 1):
      issue_idx(b, b)

    pltpu.sync_copy(zrows_hbm, rows[0].at[pl.ds(0, RCH)])

    def zero_chunk(jj, carry):
      j = s + NS * jj

      @pl.when(j < NRCH)
      def _():
        pltpu.sync_copy(rows[0].at[pl.ds(0, RCH)], acc.at[pl.ds(j * RCH, RCH)])

      return carry

    lax.fori_loop(0, RROUND, zero_chunk, 0)

    for b in range(GRING - 1):
      issue_gather(b, b)
    plsc.subcore_barrier()

    def step(jj, carry):
      for b in range(IRING):
        jc = jj * IRING + b
        gb = b % GRING

        @pl.when(jc + IRING - 1 < NCH)
        def _(jc=jc, b=b):
          issue_idx(jc + IRING - 1, (b + IRING - 1) % IRING)

        @pl.when(jc + GRING - 1 < NCH)
        def _(jc=jc, b=b, gb=gb):
          issue_gather((b + GRING - 1) % IRING, (gb + GRING - 1) % GRING)

        @pl.when(jc < NCH)
        def _(jc=jc, b=b, gb=gb):
          pltpu.make_async_copy(x_hbm.at[sidx[b]], rows[gb], gsem[gb]).wait()
          pltpu.make_async_copy(dst_hbm.at[pl.ds(0, K)], didx[b],
                                dsem[b]).wait()
          pltpu.sync_copy(rows[gb], acc.at[didx[b]], add=True)

      return carry

    lax.fori_loop(0, -(-NCH // IRING), step, 0)
    plsc.subcore_barrier()

    def wb_chunk(jj, carry):
      j = s + NS * jj

      @pl.when(j < NRCH)
      def _():
        r0 = j * RCH
        pltpu.sync_copy(acc.at[pl.ds(r0, RCH)], rows[0].at[pl.ds(0, RCH)])
        pltpu.sync_copy(rows[0].at[pl.ds(0, RCH)],
                        out_hbm.at[c, pl.ds(r0, RCH)])

      return carry

    lax.fori_loop(0, RROUND, wb_chunk, 0)

  return seg


_seg128 = _seg_partials(H)


def _fused_mm_body(x_ref, p_ref, b_ref, w_ref, o_ref, *, relu):
  z = x_ref[...] + p_ref[0] + p_ref[1]
  y = jnp.dot(z, w_ref[...], preferred_element_type=jnp.float32) + b_ref[...]
  o_ref[...] = jnp.maximum(y, 0.0) if relu else y


def _fused_mm(x, p, b, w, relu):
  nblk = N // _MM_BLK
  din = x.shape[1]
  dout = w.shape[1]
  return pl.pallas_call(
      functools.partial(_fused_mm_body, relu=relu),
      grid=(nblk,),
      in_specs=[
          pl.BlockSpec((_MM_BLK, din), lambda i: (i, 0)),
          pl.BlockSpec((NC, _MM_BLK, din), lambda i: (0, i, 0)),
          pl.BlockSpec((1, dout), lambda i: (0, 0)),
          pl.BlockSpec((din, dout), lambda i: (0, 0)),
      ],
      out_specs=pl.BlockSpec((_MM_BLK, dout), lambda i: (i, 0)),
      out_shape=jax.ShapeDtypeStruct((N, dout), jnp.float32),
  )(x, p, b.reshape(1, dout), w)


def kernel(features, adj, W1, b1, W2, b2):
  src = adj[0]
  dst = adj[1]
  zrows = jnp.zeros((RCH, H), jnp.float32)

  p1 = _seg128(features, src, dst, zrows)
  z1 = _fused_mm(features, p1, b1, W1, relu=True)
  p2 = _seg128(z1, src, dst, zrows)
  out = _fused_mm(z1, p2, b2, W2, relu=False)
  return out


# speedup vs baseline: 1.0855x; 1.0098x over previous
import functools

import jax
import jax.numpy as jnp
from jax import lax
from jax.experimental import pallas as pl
from jax.experimental.pallas import tpu as pltpu
from jax.experimental.pallas import tpu_sc as plsc

N = 10000
E = 320000
D = 128
H = 128
C = 64

NC = 2
NS = 16
NW = NC * NS

EPT = E // NW
K = 80
NCH = EPT // K
IRING = 8
GRING = 4
RCH = 80
NRCH = N // RCH
RROUND = -(-NRCH // NS)

_MM_BLK = 1000


def _seg_partials(F):
  mesh = plsc.VectorSubcoreMesh(core_axis_name="c", subcore_axis_name="s")

  scratch = []
  for _ in range(IRING):
    scratch += [
        pltpu.VMEM((K,), jnp.int32),
        pltpu.VMEM((K,), jnp.int32),
        pltpu.SemaphoreType.DMA,
        pltpu.SemaphoreType.DMA,
    ]
  for _ in range(GRING):
    scratch += [
        pltpu.VMEM((K, F), jnp.float32),
        pltpu.SemaphoreType.DMA,
    ]
  scratch.append(pltpu.VMEM_SHARED((N, F), jnp.float32))

  @functools.partial(
      pl.kernel,
      out_type=jax.ShapeDtypeStruct((NC, N, F), jnp.float32),
      mesh=mesh,
      scratch_types=scratch,
  )
  def seg(x_hbm, src_hbm, dst_hbm, zrows_hbm, out_hbm, *bufs):
    sidx = [bufs[4 * b + 0] for b in range(IRING)]
    didx = [bufs[4 * b + 1] for b in range(IRING)]
    ssem = [bufs[4 * b + 2] for b in range(IRING)]
    dsem = [bufs[4 * b + 3] for b in range(IRING)]
    g0 = 4 * IRING
    rows = [bufs[g0 + 2 * b] for b in range(GRING)]
    gsem = [bufs[g0 + 2 * b + 1] for b in range(GRING)]
    acc = bufs[-1]

    c = lax.axis_index("c")
    s = lax.axis_index("s")
    wid = s * NC + c
    ebase = wid * EPT

    def issue_idx(jc, ib):
      off = ebase + jc * K
      pltpu.async_copy(src_hbm.at[pl.ds(off, K)], sidx[ib], ssem[ib])
      pltpu.async_copy(dst_hbm.at[pl.ds(off, K)], didx[ib], dsem[ib])

    def issue_gather(ib, gb):
      pltpu.make_async_copy(src_hbm.at[pl.ds(0, K)], sidx[ib],
                            ssem[ib]).wait()
      pltpu.async_copy(x_hbm.at[sidx[ib]], rows[gb], gsem[gb])

    for b in range(IRING - 1):
      issue_idx(b, b)

    pltpu.sync_copy(zrows_hbm, rows[0].at[pl.ds(0, RCH)])

    def zero_issue(jj, carry):
      j = s + NS * jj

      @pl.when(j < NRCH)
      def _():
        pltpu.async_copy(rows[0].at[pl.ds(0, RCH)],
                         acc.at[pl.ds(j * RCH, RCH)], gsem[GRING - 1])

      return carry

    def zero_drain(jj, carry):
      j = s + NS * jj

      @pl.when(j < NRCH)
      def _():
        pltpu.make_async_copy(rows[0].at[pl.ds(0, RCH)],
                              acc.at[pl.ds(0, RCH)], gsem[GRING - 1]).wait()

      return carry

    lax.fori_loop(0, RROUND, zero_issue, 0)
    lax.fori_loop(0, RROUND, zero_drain, 0)

    for b in range(GRING - 1):
      issue_gather(b, b)
    plsc.subcore_barrier()

    def step(jj, carry):
      for b in range(IRING):
        jc = jj * IRING + b
        gb = b % GRING

        @pl.when(jc + IRING - 1 < NCH)
        def _(jc=jc, b=b):
          issue_idx(jc + IRING - 1, (b + IRING - 1) % IRING)

        @pl.when(jc + GRING - 1 < NCH)
        def _(jc=jc, b=b, gb=gb):
          issue_gather((b + GRING - 1) % IRING, (gb + GRING - 1) % GRING)

        @pl.when(jc < NCH)
        def _(jc=jc, b=b, gb=gb):
          pltpu.make_async_copy(x_hbm.at[sidx[b]], rows[gb], gsem[gb]).wait()
          pltpu.make_async_copy(dst_hbm.at[pl.ds(0, K)], didx[b],
                                dsem[b]).wait()
          pltpu.sync_copy(rows[gb], acc.at[didx[b]], add=True)

      return carry

    lax.fori_loop(0, -(-NCH // IRING), step, 0)
    plsc.subcore_barrier()

    def wb_issue(jj, carry):
      j = s + NS * jj

      @pl.when(j < NRCH)
      def _():
        r0 = j * RCH
        pltpu.async_copy(acc.at[pl.ds(r0, RCH)],
                         out_hbm.at[c, pl.ds(r0, RCH)], gsem[0])

      return carry

    def wb_drain(jj, carry):
      j = s + NS * jj

      @pl.when(j < NRCH)
      def _():
        pltpu.make_async_copy(acc.at[pl.ds(0, RCH)],
                              out_hbm.at[c, pl.ds(0, RCH)], gsem[0]).wait()

      return carry

    lax.fori_loop(0, RROUND, wb_issue, 0)
    lax.fori_loop(0, RROUND, wb_drain, 0)

  return seg


_seg128 = _seg_partials(H)


def _fused_mm_body(x_ref, p_ref, b_ref, w_ref, o_ref, *, relu):
  z = x_ref[...] + p_ref[0] + p_ref[1]
  y = jnp.dot(z, w_ref[...], preferred_element_type=jnp.float32) + b_ref[...]
  o_ref[...] = jnp.maximum(y, 0.0) if relu else y


def _fused_mm(x, p, b, w, relu):
  nblk = N // _MM_BLK
  din = x.shape[1]
  dout = w.shape[1]
  return pl.pallas_call(
      functools.partial(_fused_mm_body, relu=relu),
      grid=(nblk,),
      in_specs=[
          pl.BlockSpec((_MM_BLK, din), lambda i: (i, 0)),
          pl.BlockSpec((NC, _MM_BLK, din), lambda i: (0, i, 0)),
          pl.BlockSpec((1, dout), lambda i: (0, 0)),
          pl.BlockSpec((din, dout), lambda i: (0, 0)),
      ],
      out_specs=pl.BlockSpec((_MM_BLK, dout), lambda i: (i, 0)),
      out_shape=jax.ShapeDtypeStruct((N, dout), jnp.float32),
  )(x, p, b.reshape(1, dout), w)


def kernel(features, adj, W1, b1, W2, b2):
  src = adj[0]
  dst = adj[1]
  zrows = jnp.zeros((RCH, H), jnp.float32)

  p1 = _seg128(features, src, dst, zrows)
  z1 = _fused_mm(features, p1, b1, W1, relu=True)
  p2 = _seg128(z1, src, dst, zrows)
  out = _fused_mm(z1, p2, b2, W2, relu=False)
  return out


# speedup vs baseline: 1.0989x; 1.0124x over previous
import functools

import jax
import jax.numpy as jnp
from jax import lax
from jax.experimental import pallas as pl
from jax.experimental.pallas import tpu as pltpu
from jax.experimental.pallas import tpu_sc as plsc

N = 10000
E = 320000
D = 128
H = 128
C = 64

NC = 2
NS = 16
NW = NC * NS

EPT = E // NW
K = 80
NCH = EPT // K
IRING = 8
GRING = 4
RCH = 80
NRCH = N // RCH
RROUND = -(-NRCH // NS)

_MM_BLK = 1000


def _seg_partials(F):
  mesh = plsc.VectorSubcoreMesh(core_axis_name="c", subcore_axis_name="s")

  scratch = []
  for _ in range(IRING):
    scratch += [
        pltpu.VMEM((K,), jnp.int32),
        pltpu.VMEM((K,), jnp.int32),
        pltpu.SemaphoreType.DMA,
        pltpu.SemaphoreType.DMA,
    ]
  for _ in range(GRING):
    scratch += [
        pltpu.VMEM((K, F), jnp.float32),
        pltpu.SemaphoreType.DMA,
    ]
  scratch.append(pltpu.VMEM_SHARED((N, F), jnp.float32))

  @functools.partial(
      pl.kernel,
      out_type=jax.ShapeDtypeStruct((NC, N, F), jnp.float32),
      mesh=mesh,
      scratch_types=scratch,
  )
  def seg(x_hbm, src_hbm, dst_hbm, zrows_hbm, out_hbm, *bufs):
    sidx = [bufs[4 * b + 0] for b in range(IRING)]
    didx = [bufs[4 * b + 1] for b in range(IRING)]
    ssem = [bufs[4 * b + 2] for b in range(IRING)]
    dsem = [bufs[4 * b + 3] for b in range(IRING)]
    g0 = 4 * IRING
    rows = [bufs[g0 + 2 * b] for b in range(GRING)]
    gsem = [bufs[g0 + 2 * b + 1] for b in range(GRING)]
    acc = bufs[-1]

    c = lax.axis_index("c")
    s = lax.axis_index("s")
    wid = s * NC + c
    ebase = wid * EPT

    def issue_idx(jc, ib):
      off = ebase + jc * K
      pltpu.async_copy(src_hbm.at[pl.ds(off, K)], sidx[ib], ssem[ib])
      pltpu.async_copy(dst_hbm.at[pl.ds(off, K)], didx[ib], dsem[ib])

    def issue_gather(ib, gb):
      pltpu.make_async_copy(src_hbm.at[pl.ds(0, K)], sidx[ib],
                            ssem[ib]).wait()
      pltpu.async_copy(x_hbm.at[sidx[ib]], rows[gb], gsem[gb])

    for b in range(IRING - 1):
      issue_idx(b, b)

    pltpu.sync_copy(zrows_hbm, rows[GRING - 1].at[pl.ds(0, RCH)])

    def zero_issue(jj, carry):
      j = s + NS * jj

      @pl.when(j < NRCH)
      def _():
        pltpu.async_copy(rows[GRING - 1].at[pl.ds(0, RCH)],
                         acc.at[pl.ds(j * RCH, RCH)], gsem[GRING - 1])

      return carry

    def zero_drain(jj, carry):
      j = s + NS * jj

      @pl.when(j < NRCH)
      def _():
        pltpu.make_async_copy(rows[GRING - 1].at[pl.ds(0, RCH)],
                              acc.at[pl.ds(0, RCH)], gsem[GRING - 1]).wait()

      return carry

    lax.fori_loop(0, RROUND, zero_issue, 0)
    for b in range(GRING - 1):
      issue_gather(b, b)
    lax.fori_loop(0, RROUND, zero_drain, 0)
    plsc.subcore_barrier()

    def step(jj, carry):
      for b in range(IRING):
        jc = jj * IRING + b
        gb = b % GRING

        @pl.when(jc + IRING - 1 < NCH)
        def _(jc=jc, b=b):
          issue_idx(jc + IRING - 1, (b + IRING - 1) % IRING)

        @pl.when(jc + GRING - 1 < NCH)
        def _(jc=jc, b=b, gb=gb):
          issue_gather((b + GRING - 1) % IRING, (gb + GRING - 1) % GRING)

        @pl.when(jc < NCH)
        def _(jc=jc, b=b, gb=gb):
          pltpu.make_async_copy(x_hbm.at[sidx[b]], rows[gb], gsem[gb]).wait()
          pltpu.make_async_copy(dst_hbm.at[pl.ds(0, K)], didx[b],
                                dsem[b]).wait()
          pltpu.sync_copy(rows[gb], acc.at[didx[b]], add=True)

      return carry

    lax.fori_loop(0, -(-NCH // IRING), step, 0)
    plsc.subcore_barrier()

    def wb_issue(jj, carry):
      j = s + NS * jj

      @pl.when(j < NRCH)
      def _():
        r0 = j * RCH
        pltpu.async_copy(acc.at[pl.ds(r0, RCH)],
                         out_hbm.at[c, pl.ds(r0, RCH)], gsem[0])

      return carry

    def wb_drain(jj, carry):
      j = s + NS * jj

      @pl.when(j < NRCH)
      def _():
        pltpu.make_async_copy(acc.at[pl.ds(0, RCH)],
                              out_hbm.at[c, pl.ds(0, RCH)], gsem[0]).wait()

      return carry

    lax.fori_loop(0, RROUND, wb_issue, 0)
    lax.fori_loop(0, RROUND, wb_drain, 0)

  return seg


_seg128 = _seg_partials(H)


def _fused_mm_body(x_ref, p_ref, b_ref, w_ref, o_ref, *, relu):
  z = x_ref[...] + p_ref[0] + p_ref[1]
  y = jnp.dot(z, w_ref[...], preferred_element_type=jnp.float32) + b_ref[...]
  o_ref[...] = jnp.maximum(y, 0.0) if relu else y


def _fused_mm(x, p, b, w, relu):
  nblk = N // _MM_BLK
  din = x.shape[1]
  dout = w.shape[1]
  return pl.pallas_call(
      functools.partial(_fused_mm_body, relu=relu),
      grid=(nblk,),
      in_specs=[
          pl.BlockSpec((_MM_BLK, din), lambda i: (i, 0)),
          pl.BlockSpec((NC, _MM_BLK, din), lambda i: (0, i, 0)),
          pl.BlockSpec((1, dout), lambda i: (0, 0)),
          pl.BlockSpec((din, dout), lambda i: (0, 0)),
      ],
      out_specs=pl.BlockSpec((_MM_BLK, dout), lambda i: (i, 0)),
      out_shape=jax.ShapeDtypeStruct((N, dout), jnp.float32),
  )(x, p, b.reshape(1, dout), w)


def kernel(features, adj, W1, b1, W2, b2):
  src = adj[0]
  dst = adj[1]
  zrows = jnp.zeros((RCH, H), jnp.float32)

  p1 = _seg128(features, src, dst, zrows)
  z1 = _fused_mm(features, p1, b1, W1, relu=True)
  p2 = _seg128(z1, src, dst, zrows)
  out = _fused_mm(z1, p2, b2, W2, relu=False)
  return out
